# Initial kernel scaffold; baseline (speedup 1.0000x reference)
import jax
import jax.numpy as jnp
from jax.experimental import pallas as pl


def kernel(x, edge_index, W_l1, b_l1, W_r1, W_l2, b_l2, W_r2):
    raise NotImplementedError("write your pallas kernel here")


# speedup vs baseline: 3.7269x; 3.7269x over previous
import functools

import jax
import jax.numpy as jnp
from jax import lax
from jax.experimental import pallas as pl
from jax.experimental.pallas import tpu as pltpu
from jax.experimental.pallas import tpu_sc as plsc

N = 10000
E = 320000
D = 128
NC = 2
NS = 16
NW = NC * NS
RPW = 80
EROWS = NW * RPW
ACC_N = N + 16
NPT = 624
ACC1 = 10240
NPT1 = ACC1 // NS
CH = 8
NCHUNK = RPW // CH

def _sc_segsum_body(with_counts, *refs):
    if with_counts:
        (u_hbm, srcr, dstr, z128, z1,
         s_out, c_out,
         s_sh, c_sh, src_v, dst_v, rows_v, ones_v, sem) = refs
    else:
        (u_hbm, srcr, dstr, z128,
         s_out,
         s_sh, src_v, dst_v, rows_v, sem) = refs

    cid = lax.axis_index("c")
    sid = lax.axis_index("s")
    wid = cid * NS + sid

    pltpu.sync_copy(z128.at[pl.ds(sid * NPT, NPT)],
                    s_sh.at[pl.ds(sid * NPT, NPT)])
    if with_counts:
        pltpu.sync_copy(z1.at[pl.ds(sid * NPT1, NPT1)],
                        c_sh.at[pl.ds(sid * NPT1, NPT1)])
        for k in range(D // 16):
            ones_v[pl.ds(k * 16, 16)] = jnp.ones((16,), jnp.float32)

    @pl.when(sid == NS - 1)
    def _():
        tail = ACC_N - NS * NPT
        pltpu.sync_copy(z128.at[pl.ds(0, tail)],
                        s_sh.at[pl.ds(NS * NPT, tail)])

    base = wid * RPW
    pltpu.sync_copy(srcr.at[pl.ds(base, CH)], src_v.at[0])
    pltpu.sync_copy(dstr.at[pl.ds(base, CH)], dst_v.at[0])

    plsc.subcore_barrier()

    pltpu.async_copy(u_hbm.at[src_v.at[0].at[0]], rows_v.at[0], sem)

    def step(j, carry):
        c = j // CH
        r = j - c * CH

        @pl.when((r == 0) & (c + 1 < NCHUNK))
        def _():
            pltpu.sync_copy(srcr.at[pl.ds(base + (c + 1) * CH, CH)],
                            src_v.at[(c + 1) % 2])
            pltpu.sync_copy(dstr.at[pl.ds(base + (c + 1) * CH, CH)],
                            dst_v.at[(c + 1) % 2])

        nxt = j + 1

        @pl.when(nxt < RPW)
        def _():
            c2 = nxt // CH
            r2 = nxt - c2 * CH
            pltpu.async_copy(u_hbm.at[src_v.at[c2 % 2].at[r2]],
                             rows_v.at[nxt % 2], sem)

        pltpu.make_async_copy(u_hbm.at[src_v.at[c % 2].at[r]],
                              rows_v.at[j % 2], sem).wait()
        pltpu.sync_copy(rows_v.at[j % 2], s_sh.at[dst_v.at[c % 2].at[r]],
                        add=True)
        if with_counts:
            pltpu.sync_copy(ones_v, c_sh.at[dst_v.at[c % 2].at[r]], add=True)
        return carry

    lax.fori_loop(0, RPW, step, 0)

    plsc.subcore_barrier()

    pltpu.sync_copy(s_sh.at[pl.ds(sid * NPT, NPT)],
                    s_out.at[cid].at[pl.ds(sid * NPT, NPT)])
    if with_counts:
        pltpu.sync_copy(c_sh.at[pl.ds(sid * NPT1, NPT1)],
                        c_out.at[cid].at[pl.ds(sid * NPT1, NPT1)])

    @pl.when(sid == NS - 1)
    def _():
        tail = N - NS * NPT
        pltpu.sync_copy(s_sh.at[pl.ds(NS * NPT, tail)],
                        s_out.at[cid].at[pl.ds(NS * NPT, tail)])


@functools.cache
def _sc_kernels():
    mesh = plsc.VectorSubcoreMesh(
        core_axis_name="c", subcore_axis_name="s",
        num_cores=NC, num_subcores=NS,
    )
    layer1 = functools.partial(
        pl.kernel,
        functools.partial(_sc_segsum_body, True),
        out_type=(
            jax.ShapeDtypeStruct((NC, N, D), jnp.float32),
            jax.ShapeDtypeStruct((NC, ACC1), jnp.float32),
        ),
        mesh=mesh,
        scratch_types=[
            pltpu.VMEM_SHARED((ACC_N, D), jnp.float32),
            pltpu.VMEM_SHARED((ACC1,), jnp.float32),
            pltpu.VMEM((2, CH, D), jnp.int32),
            pltpu.VMEM((2, CH, D), jnp.int32),
            pltpu.VMEM((2, D, D), jnp.float32),
            pltpu.VMEM((D,), jnp.float32),
            pltpu.SemaphoreType.DMA,
        ],
    )()
    layer2 = functools.partial(
        pl.kernel,
        functools.partial(_sc_segsum_body, False),
        out_type=jax.ShapeDtypeStruct((NC, N, D), jnp.float32),
        mesh=mesh,
        scratch_types=[
            pltpu.VMEM_SHARED((ACC_N, D), jnp.float32),
            pltpu.VMEM((2, CH, D), jnp.int32),
            pltpu.VMEM((2, CH, D), jnp.int32),
            pltpu.VMEM((2, D, D), jnp.float32),
            pltpu.SemaphoreType.DMA,
        ],
    )()
    return layer1, layer2


_RB = 1000


def _tc_pre_body(x_ref, wl_ref, b_ref, wr_ref, u_ref, v_ref):
    x = x_ref[...]
    u_ref[...] = jnp.dot(x, wl_ref[...].T, preferred_element_type=jnp.float32)
    v_ref[...] = (jnp.dot(x, wr_ref[...].T, preferred_element_type=jnp.float32)
                  + b_ref[...])


def _tc_mid_body(s_ref, c_ref, v1_ref, wl_ref, b_ref, wr_ref, u2_ref, v2_ref):
    cnt = c_ref[...]
    inv = 1.0 / jnp.maximum(cnt, 1.0)
    h = jnp.maximum((s_ref[0] + s_ref[1]) * inv + v1_ref[...], 0.0)
    u2_ref[...] = jnp.dot(h, wl_ref[...].T, preferred_element_type=jnp.float32)
    v2_ref[...] = (jnp.dot(h, wr_ref[...].T, preferred_element_type=jnp.float32)
                   + b_ref[...])


def _tc_post_body(s_ref, c_ref, v2_ref, out_ref):
    cnt = c_ref[...]
    inv = 1.0 / jnp.maximum(cnt, 1.0)
    out_ref[...] = (s_ref[0] + s_ref[1]) * inv + v2_ref[...]


def _full(shape):
    return pl.BlockSpec(shape, lambda i: (0,) * len(shape))


def _rows(shape):
    if len(shape) == 3:
        return pl.BlockSpec(shape, lambda i: (0, i, 0))
    return pl.BlockSpec(shape, lambda i: (i, 0))


_tc_pre = pl.pallas_call(
    _tc_pre_body,
    grid=(N // _RB,),
    in_specs=[_rows((_RB, D)), _full((D, D)), _full((1, D)), _full((D, D))],
    out_specs=[_rows((_RB, D)), _rows((_RB, D))],
    out_shape=(jax.ShapeDtypeStruct((N, D), jnp.float32),
               jax.ShapeDtypeStruct((N, D), jnp.float32)),
)

_tc_mid = pl.pallas_call(
    _tc_mid_body,
    grid=(N // _RB,),
    in_specs=[_rows((NC, _RB, D)), _rows((_RB, 1)), _rows((_RB, D)),
              _full((D, D)), _full((1, D)), _full((D, D))],
    out_specs=[_rows((_RB, D)), _rows((_RB, D))],
    out_shape=(jax.ShapeDtypeStruct((N, D), jnp.float32),
               jax.ShapeDtypeStruct((N, D), jnp.float32)),
)

_tc_post = pl.pallas_call(
    _tc_post_body,
    grid=(N // _RB,),
    in_specs=[_rows((NC, _RB, D)), _rows((_RB, 1)), _rows((_RB, D))],
    out_specs=_rows((_RB, D)),
    out_shape=jax.ShapeDtypeStruct((N, D), jnp.float32),
)


@jax.jit
def kernel(x, edge_index, W_l1, b_l1, W_r1, W_l2, b_l2, W_r2):
    pad = EROWS * D - E
    srcr = jnp.concatenate(
        [edge_index[0].astype(jnp.int32), jnp.zeros((pad,), jnp.int32)]
    ).reshape(EROWS, D)
    dstr = jnp.concatenate(
        [edge_index[1].astype(jnp.int32), jnp.full((pad,), N, jnp.int32)]
    ).reshape(EROWS, D)
    z128 = jnp.zeros((N, D), jnp.float32)
    z1 = jnp.zeros((ACC1,), jnp.float32)

    sc_layer1, sc_layer2 = _sc_kernels()
    u1, v1 = _tc_pre(x, W_l1, b_l1.reshape(1, D), W_r1)
    s1, cpart = sc_layer1(u1, srcr, dstr, z128, z1)
    cnt_t = (cpart[0, :N] + cpart[1, :N]).reshape(N, 1)
    u2, v2 = _tc_mid(s1, cnt_t, v1, W_l2, b_l2.reshape(1, D), W_r2)
    s2 = sc_layer2(u2, srcr, dstr, z128)
    return _tc_post(s2, cnt_t, v2)
